# baseline (speedup 1.0000x reference)
import jax
import jax.numpy as jnp
from jax.experimental import pallas as pl
from jax.experimental.pallas import tpu as pltpu

_B, _D, _E = 8192, 4096, 64
_BLK = 1024
_NBLK = _B // _BLK
_KP = 4
_KC = _D // _KP


def _leaky(h):
    return jnp.where(h >= 0, h, 0.01 * h)


def _gate_kernel(x_ref, w1_ref, b1_ref, w2_ref, b2_ref, w3_ref, b3_ref,
                 w4_ref, b4_ref, out_ref, h_acc_ref, sum_ref):
    i = pl.program_id(0)
    j = pl.program_id(1)
    nblk = pl.num_programs(0)

    @pl.when((i == 0) & (j == 0))
    def _init():
        sum_ref[0] = 0.0

    part = jnp.dot(x_ref[...], w1_ref[pl.ds(j * _KC, _KC), :],
                   preferred_element_type=jnp.float32)

    @pl.when(j == 0)
    def _first_phase():
        h_acc_ref[...] = part

    @pl.when(j > 0)
    def _other_phase():
        h_acc_ref[...] += part

    @pl.when(j == _KP - 1)
    def _finish_block():
        h = jnp.maximum(h_acc_ref[...] + b1_ref[...], 0.0)
        h = _leaky(jnp.dot(h, w2_ref[...], preferred_element_type=jnp.float32)
                   + b2_ref[...])
        h = _leaky(jnp.dot(h, w3_ref[...], preferred_element_type=jnp.float32)
                   + b3_ref[...])
        logits = (jnp.dot(h, w4_ref[...], preferred_element_type=jnp.float32)
                  + b4_ref[...])

        m1 = jnp.max(logits, axis=1, keepdims=True)
        is_max = logits == m1
        dup = jnp.sum(is_max.astype(jnp.float32), axis=1, keepdims=True) > 1.0
        m2_lo = jnp.max(jnp.where(is_max, -jnp.inf, logits),
                        axis=1, keepdims=True)
        m2 = jnp.where(dup, m1, m2_lo)
        sum_ref[0] += jnp.sum(m1) + jnp.sum(m2)

        @pl.when(i < nblk - 1)
        def _store_zeros():
            out_ref[...] = jnp.zeros_like(logits)

        @pl.when(i == nblk - 1)
        def _store_final():
            s = sum_ref[0]
            col = jax.lax.broadcasted_iota(jnp.int32, logits.shape, 1)
            a1 = jnp.min(jnp.where(is_max, col, _E), axis=1, keepdims=True)
            masked = jnp.where(col == a1, -jnp.inf, logits)
            a2 = jnp.min(jnp.where(masked == m2, col, _E),
                         axis=1, keepdims=True)
            row = jax.lax.broadcasted_iota(jnp.int32, logits.shape, 0)
            vals = jnp.where(col == a1, m1 / s,
                             jnp.where(col == a2, m2 / s, 0.0))
            out_ref[...] = jnp.where(row == 0, vals, 0.0)


def kernel(x, W1, b1, W2, b2, W3, b3, W4, b4):
    w1t, w2t, w3t, w4t = W1.T, W2.T, W3.T, W4.T
    b1r, b2r, b3r, b4r = (b.reshape(1, -1) for b in (b1, b2, b3, b4))

    full = lambda shape: pl.BlockSpec(shape, lambda i, j: (0, 0))
    return pl.pallas_call(
        _gate_kernel,
        grid=(_NBLK, _KP),
        in_specs=[
            pl.BlockSpec((_BLK, _KC), lambda i, j: (_NBLK - 1 - i, j)),
            full((_D, 128)), full((1, 128)),
            full((128, 256)), full((1, 256)),
            full((256, 128)), full((1, 128)),
            full((128, _E)), full((1, _E)),
        ],
        out_specs=pl.BlockSpec((_BLK, _E), lambda i, j: (_NBLK - 1 - i, 0)),
        out_shape=jax.ShapeDtypeStruct((_B, _E), jnp.float32),
        scratch_shapes=[pltpu.VMEM((_BLK, 128), jnp.float32),
                        pltpu.SMEM((1,), jnp.float32)],
    )(x, w1t, b1r, w2t, b2r, w3t, b3r, w4t, b4r)


# speedup vs baseline: 1.1511x; 1.1511x over previous
import jax
import jax.numpy as jnp
from jax.experimental import pallas as pl
from jax.experimental.pallas import tpu as pltpu

_B, _D, _E = 8192, 4096, 64
_BLK = 1024
_NBLK = _B // _BLK


def _leaky(h):
    return jnp.where(h >= 0, h, 0.01 * h)


def _gate_kernel(x_ref, w1_hbm, b1_hbm, w2_hbm, b2_hbm, w3_hbm, b3_hbm,
                 w4_hbm, b4_hbm, out_ref,
                 w1_v, b1_v, w2_v, b2_v, w3_v, b3_v, w4_v, b4_v,
                 copy_sem, acc_ref):
    i = pl.program_id(0)
    nsteps = pl.num_programs(0)

    @pl.when(i == 0)
    def _init():
        acc_ref[0] = 0.0
        copies = [
            pltpu.make_async_copy(w1_hbm, w1_v, copy_sem),
            pltpu.make_async_copy(b1_hbm, b1_v, copy_sem),
            pltpu.make_async_copy(w2_hbm, w2_v, copy_sem),
            pltpu.make_async_copy(b2_hbm, b2_v, copy_sem),
            pltpu.make_async_copy(w3_hbm, w3_v, copy_sem),
            pltpu.make_async_copy(b3_hbm, b3_v, copy_sem),
            pltpu.make_async_copy(w4_hbm, w4_v, copy_sem),
            pltpu.make_async_copy(b4_hbm, b4_v, copy_sem),
        ]
        for c in copies:
            c.start()
        for c in copies:
            c.wait()

    h = jnp.dot(x_ref[...], w1_v[...], preferred_element_type=jnp.float32)
    h = jnp.maximum(h + b1_v[...], 0.0)
    h = _leaky(jnp.dot(h, w2_v[...], preferred_element_type=jnp.float32)
               + b2_v[...])
    h = _leaky(jnp.dot(h, w3_v[...], preferred_element_type=jnp.float32)
               + b3_v[...])
    logits = (jnp.dot(h, w4_v[...], preferred_element_type=jnp.float32)
              + b4_v[...])

    m1 = jnp.max(logits, axis=1, keepdims=True)
    is_max = logits == m1
    dup = jnp.sum(is_max.astype(jnp.float32), axis=1, keepdims=True) > 1.0
    m2_lo = jnp.max(jnp.where(is_max, -jnp.inf, logits), axis=1, keepdims=True)
    m2 = jnp.where(dup, m1, m2_lo)
    acc_ref[0] += jnp.sum(m1) + jnp.sum(m2)

    @pl.when(i < nsteps - 1)
    def _store_zeros():
        out_ref[...] = jnp.zeros_like(logits)

    @pl.when(i == nsteps - 1)
    def _store_final():
        s = acc_ref[0]
        col = jax.lax.broadcasted_iota(jnp.int32, logits.shape, 1)
        a1 = jnp.min(jnp.where(is_max, col, _E), axis=1, keepdims=True)
        masked = jnp.where(col == a1, -jnp.inf, logits)
        a2 = jnp.min(jnp.where(masked == m2, col, _E), axis=1, keepdims=True)
        row = jax.lax.broadcasted_iota(jnp.int32, logits.shape, 0)
        vals = jnp.where(col == a1, m1 / s,
                         jnp.where(col == a2, m2 / s, 0.0))
        out_ref[...] = jnp.where(row == 0, vals, 0.0)


def kernel(x, W1, b1, W2, b2, W3, b3, W4, b4):
    w1t, w2t, w3t, w4t = W1.T, W2.T, W3.T, W4.T
    b1r, b2r, b3r, b4r = (b.reshape(1, -1) for b in (b1, b2, b3, b4))

    anyspec = pl.BlockSpec(memory_space=pltpu.MemorySpace.HBM)
    return pl.pallas_call(
        _gate_kernel,
        grid=(_NBLK,),
        in_specs=[
            pl.BlockSpec((_BLK, _D), lambda i: (_NBLK - 1 - i, 0)),
        ] + [anyspec] * 8,
        out_specs=pl.BlockSpec((_BLK, _E), lambda i: (_NBLK - 1 - i, 0)),
        out_shape=jax.ShapeDtypeStruct((_B, _E), jnp.float32),
        scratch_shapes=[
            pltpu.VMEM((_D, 128), jnp.float32), pltpu.VMEM((1, 128), jnp.float32),
            pltpu.VMEM((128, 256), jnp.float32), pltpu.VMEM((1, 256), jnp.float32),
            pltpu.VMEM((256, 128), jnp.float32), pltpu.VMEM((1, 128), jnp.float32),
            pltpu.VMEM((128, _E), jnp.float32), pltpu.VMEM((1, _E), jnp.float32),
            pltpu.SemaphoreType.DMA,
            pltpu.SMEM((1,), jnp.float32),
        ],
    )(x, w1t, b1r, w2t, b2r, w3t, b3r, w4t, b4r)
